# baseline (speedup 1.0000x reference)
import jax
import jax.numpy as jnp
from jax.experimental import pallas as pl
from jax.experimental.pallas import tpu as pltpu
from jax.experimental.pallas import tpu_sc as plsc

_N = 10000
_M = 2500
_D = 128
_BM = 128
_M_PAD = 2560
_GW = 80


def _top2_body(sc_ref, ct_ref, idx_ref):
    a = sc_ref[0]
    ct = ct_ref[0]
    a0, a1, a2 = a[:, 0:1], a[:, 1:2], a[:, 2:3]
    b0, b1, b2 = ct[0:1, :], ct[1:2, :], ct[2:3, :]
    aa = (a0 * a0 + a2 * a2) + a1 * a1
    bb = (b0 * b0 + b2 * b2) + b1 * b1
    ab = jax.lax.dot_general(a, ct, (((1,), (0,)), ((), ())),
                             preferred_element_type=jnp.float32)
    dist = (aa + bb) - 2.0 * ab
    iota = jax.lax.broadcasted_iota(jnp.int32, dist.shape, 1).astype(jnp.float32)
    big = jnp.float32(_N)
    m1 = jnp.min(dist, axis=1, keepdims=True)
    i1 = jnp.min(jnp.where(dist == m1, iota, big), axis=1, keepdims=True)
    d2 = jnp.where(iota == i1, jnp.float32(jnp.inf), dist)
    m2 = jnp.min(d2, axis=1, keepdims=True)
    i2 = jnp.min(jnp.where(d2 == m2, iota, big), axis=1, keepdims=True)
    idx_ref[0] = i2.astype(jnp.int32)


def _second_nn_idx2(shortcut_pad2, ct2, interpret=False):
    return pl.pallas_call(
        _top2_body,
        grid=(2, _M_PAD // _BM),
        in_specs=[
            pl.BlockSpec((1, _BM, 3), lambda p, i: (p, i, 0)),
            pl.BlockSpec((1, 3, _N), lambda p, i: (p, 0, 0)),
        ],
        out_specs=pl.BlockSpec((1, _BM, 1), lambda p, i: (p, i, 0)),
        out_shape=jax.ShapeDtypeStruct((2, _M_PAD, 1), jnp.int32),
        interpret=interpret,
    )(shortcut_pad2, ct2)


def _sc_gather(feats, idx_2d):
    mesh = plsc.VectorSubcoreMesh(core_axis_name="c", subcore_axis_name="s")

    @pl.kernel(out_type=jax.ShapeDtypeStruct((_M_PAD, _D), feats.dtype),
               mesh=mesh)
    def kern(x_hbm, i_hbm, o_hbm):
        def body(i_vmem, o_vmem):
            pltpu.sync_copy(x_hbm.at[i_vmem.at[0]], o_vmem)

        pltpu.emit_pipeline(
            body,
            grid=(_M_PAD // _GW,),
            in_specs=[pl.BlockSpec((1, _GW), index_map=lambda i: (i, 0))],
            out_specs=[pl.BlockSpec((_GW, _D), index_map=lambda i: (i, 0))],
            core_axis_name=("c", "s"),
            dimension_semantics=(pltpu.PARALLEL,),
        )(i_hbm, o_hbm)

    return kern(feats, idx_2d)


def kernel(src, tgt, src_coords, tgt_coords,
           src_shortcut_coords, tgt_shortcut_coords):
    pad = jnp.zeros((_M_PAD - _M, 3), jnp.float32)
    scp2 = jnp.stack([
        jnp.concatenate([src_shortcut_coords, pad], axis=0),
        jnp.concatenate([tgt_shortcut_coords, pad], axis=0),
    ])
    ct2 = jnp.stack([src_coords.T, tgt_coords.T])
    idx2 = _second_nn_idx2(scp2, ct2)
    src_out = _sc_gather(src, idx2[0].reshape(_M_PAD // _GW, _GW))
    tgt_out = _sc_gather(tgt, idx2[1].reshape(_M_PAD // _GW, _GW))
    return (src_out[:_M], tgt_out[:_M])


# speedup vs baseline: 1.0545x; 1.0545x over previous
import jax
import jax.numpy as jnp
from jax.experimental import pallas as pl
from jax.experimental.pallas import tpu as pltpu
from jax.experimental.pallas import tpu_sc as plsc

_N = 10000
_M = 2500
_D = 128
_BM = 256
_M_PAD = 2560
_GW = 80


def _top2_body(sc_ref, ct_ref, idx_ref):
    a = sc_ref[0]
    ct = ct_ref[0]
    a0, a1, a2 = a[:, 0:1], a[:, 1:2], a[:, 2:3]
    b0, b1, b2 = ct[0:1, :], ct[1:2, :], ct[2:3, :]
    aa = (a0 * a0 + a2 * a2) + a1 * a1
    bb = (b0 * b0 + b2 * b2) + b1 * b1
    ab = jax.lax.dot_general(a, ct, (((1,), (0,)), ((), ())),
                             preferred_element_type=jnp.float32)
    dist = (aa + bb) - 2.0 * ab
    iota = jax.lax.broadcasted_iota(jnp.int32, dist.shape, 1).astype(jnp.float32)
    big = jnp.float32(_N)
    m1 = jnp.min(dist, axis=1, keepdims=True)
    i1 = jnp.min(jnp.where(dist == m1, iota, big), axis=1, keepdims=True)
    d2 = jnp.where(iota == i1, jnp.float32(jnp.inf), dist)
    m2 = jnp.min(d2, axis=1, keepdims=True)
    i2 = jnp.min(jnp.where(d2 == m2, iota, big), axis=1, keepdims=True)
    idx_ref[0] = i2.astype(jnp.int32)


def _second_nn_idx2(shortcut_pad2, ct2, interpret=False):
    return pl.pallas_call(
        _top2_body,
        grid=(2, _M_PAD // _BM),
        in_specs=[
            pl.BlockSpec((1, _BM, 3), lambda p, i: (p, i, 0)),
            pl.BlockSpec((1, 3, _N), lambda p, i: (p, 0, 0)),
        ],
        out_specs=pl.BlockSpec((1, _BM, 1), lambda p, i: (p, i, 0)),
        out_shape=jax.ShapeDtypeStruct((2, _M_PAD, 1), jnp.int32),
        interpret=interpret,
    )(shortcut_pad2, ct2)


def _sc_gather(feats, idx_2d):
    mesh = plsc.VectorSubcoreMesh(core_axis_name="c", subcore_axis_name="s")

    @pl.kernel(out_type=jax.ShapeDtypeStruct((_M_PAD, _D), feats.dtype),
               mesh=mesh)
    def kern(x_hbm, i_hbm, o_hbm):
        def body(i_vmem, o_vmem):
            pltpu.sync_copy(x_hbm.at[i_vmem.at[0]], o_vmem)

        pltpu.emit_pipeline(
            body,
            grid=(_M_PAD // _GW,),
            in_specs=[pl.BlockSpec((1, _GW), index_map=lambda i: (i, 0))],
            out_specs=[pl.BlockSpec((_GW, _D), index_map=lambda i: (i, 0))],
            core_axis_name=("c", "s"),
            dimension_semantics=(pltpu.PARALLEL,),
        )(i_hbm, o_hbm)

    return kern(feats, idx_2d)


def kernel(src, tgt, src_coords, tgt_coords,
           src_shortcut_coords, tgt_shortcut_coords):
    pad = jnp.zeros((_M_PAD - _M, 3), jnp.float32)
    scp2 = jnp.stack([
        jnp.concatenate([src_shortcut_coords, pad], axis=0),
        jnp.concatenate([tgt_shortcut_coords, pad], axis=0),
    ])
    ct2 = jnp.stack([src_coords.T, tgt_coords.T])
    idx2 = _second_nn_idx2(scp2, ct2)
    src_out = _sc_gather(src, idx2[0].reshape(_M_PAD // _GW, _GW))
    tgt_out = _sc_gather(tgt, idx2[1].reshape(_M_PAD // _GW, _GW))
    return (src_out[:_M], tgt_out[:_M])
